# baseline (speedup 1.0000x reference)
import jax
import jax.numpy as jnp
from jax.experimental import pallas as pl
from jax.experimental.pallas import tpu as pltpu

_LANE_ORDER = ['dep', 'dim', 'reg', 'rot', 'wh', 'hm']
_BN_EPS = 1e-5
_R = 128
_H = 128
_W = 128
_CIN = 64
_COUT = 6 * 64


def _conv_kernel(x_ref, wk_ref, t_ref, w2_ref, b2_ref, ml_ref, mr_ref,
                 dep_ref, dim_ref, hm_ref, reg_ref, rot_ref, wh_ref,
                 xs_ref):
    ri = pl.program_id(1)
    npix = _R * _W
    nband = (_R + 2) * _W

    @pl.when((ri == 0) & (pl.program_id(0) == 0))
    def _():
        xs_ref[:, 0:_W] = jnp.zeros((_CIN, _W), jnp.bfloat16)
        xs_ref[:, _W + _H * _W:] = jnp.zeros((_CIN, _W), jnp.bfloat16)

    @pl.when(ri == 0)
    def _():
        xs_ref[:, _W:_W + _H * _W] = (
            x_ref[0].reshape(_CIN, _H * _W).astype(jnp.bfloat16))

    x = xs_ref[:, pl.ds(ri * npix, nband)]

    zc = jnp.zeros((_CIN, 1), jnp.bfloat16)
    xl = jnp.concatenate([zc, x[:, :-1]], axis=1) * ml_ref[:]
    xr = jnp.concatenate([x[:, 1:], zc], axis=1) * mr_ref[:]
    stack = jnp.concatenate([xl, x, xr], axis=0)

    rhs = jnp.concatenate(
        [stack[:, dy * _W:dy * _W + npix] for dy in range(3)], axis=0)
    acc = jnp.dot(wk_ref[:], rhs, preferred_element_type=jnp.float32)

    y = jnp.maximum(acc + t_ref[:], 0.0)
    hm = jax.nn.sigmoid(
        jnp.dot(w2_ref[:], y[320:384, :],
                preferred_element_type=jnp.float32) + b2_ref[:])

    dep_ref[0] = y[0:64, :].reshape(64, _R, _W)
    dim_ref[0] = y[64:128, :].reshape(64, _R, _W)
    reg_ref[0] = y[128:192, :].reshape(64, _R, _W)
    rot_ref[0] = y[192:256, :].reshape(64, _R, _W)
    wh_ref[0] = y[256:320, :].reshape(64, _R, _W)
    hm_ref[0] = hm.reshape(3, _R, _W)


def kernel(feat16, feat8, feat4, params):
    del feat16, feat8
    b = feat4.shape[0]

    w_all = jnp.concatenate([params[n]['w1'] for n in _LANE_ORDER], axis=0)
    inv = 1.0 / jnp.sqrt(1.0 + _BN_EPS)
    s = jnp.concatenate([params[n]['gamma'] * inv for n in _LANE_ORDER])
    t = jnp.concatenate(
        [params[n]['b1'] * params[n]['gamma'] * inv + params[n]['beta']
         for n in _LANE_ORDER])
    wk = (jnp.transpose(w_all, (0, 2, 3, 1)).reshape(_COUT, 9 * _CIN)
          * s[:, None]).astype(jnp.bfloat16)
    t = t.reshape(_COUT, 1)
    w2 = params['hm']['w2'][:, :, 0, 0]
    b2 = params['hm']['b2'].reshape(3, 1)

    nband = (_R + 2) * _W
    lane = jnp.arange(nband) % _W
    ml = (lane != 0).astype(jnp.bfloat16).reshape(1, nband)
    mr = (lane != _W - 1).astype(jnp.bfloat16).reshape(1, nband)

    nb = _H // _R
    grid = (b, nb)
    out_shapes = [
        jax.ShapeDtypeStruct((b, 64, _H, _W), jnp.float32),
        jax.ShapeDtypeStruct((b, 64, _H, _W), jnp.float32),
        jax.ShapeDtypeStruct((b, 3, _H, _W), jnp.float32),
        jax.ShapeDtypeStruct((b, 64, _H, _W), jnp.float32),
        jax.ShapeDtypeStruct((b, 64, _H, _W), jnp.float32),
        jax.ShapeDtypeStruct((b, 64, _H, _W), jnp.float32),
    ]
    head_spec = pl.BlockSpec((1, 64, _R, _W), lambda bi, ri: (bi, 0, ri, 0))
    hm_spec = pl.BlockSpec((1, 3, _R, _W), lambda bi, ri: (bi, 0, ri, 0))
    return pl.pallas_call(
        _conv_kernel,
        grid=grid,
        in_specs=[
            pl.BlockSpec((1, _CIN, _H, _W), lambda bi, ri: (bi, 0, 0, 0)),
            pl.BlockSpec((_COUT, 9 * _CIN), lambda bi, ri: (0, 0)),
            pl.BlockSpec((_COUT, 1), lambda bi, ri: (0, 0)),
            pl.BlockSpec((3, _CIN), lambda bi, ri: (0, 0)),
            pl.BlockSpec((3, 1), lambda bi, ri: (0, 0)),
            pl.BlockSpec((1, (_R + 2) * _W), lambda bi, ri: (0, 0)),
            pl.BlockSpec((1, (_R + 2) * _W), lambda bi, ri: (0, 0)),
        ],
        out_specs=[head_spec, head_spec, hm_spec, head_spec, head_spec,
                   head_spec],
        out_shape=out_shapes,
        scratch_shapes=[pltpu.VMEM((_CIN, (_H + 2) * _W), jnp.bfloat16)],
    )(feat4, wk, t, w2, b2, ml, mr)


# speedup vs baseline: 1.0265x; 1.0265x over previous
import jax
import jax.numpy as jnp
from jax.experimental import pallas as pl
from jax.experimental.pallas import tpu as pltpu

_LANE_ORDER = ['dep', 'dim', 'reg', 'rot', 'wh', 'hm']
_BN_EPS = 1e-5
_R = 64
_H = 128
_W = 128
_CIN = 64
_COUT = 6 * 64


def _conv_kernel(x_ref, wk_ref, t_ref, w2_ref, b2_ref, ml_ref, mr_ref,
                 dep_ref, dim_ref, hm_ref, reg_ref, rot_ref, wh_ref,
                 xs_ref):
    ri = pl.program_id(1)
    npix = _R * _W
    nband = (_R + 2) * _W

    @pl.when((ri == 0) & (pl.program_id(0) == 0))
    def _():
        xs_ref[:, 0:_W] = jnp.zeros((_CIN, _W), jnp.bfloat16)
        xs_ref[:, _W + _H * _W:] = jnp.zeros((_CIN, _W), jnp.bfloat16)

    @pl.when(ri == 0)
    def _():
        xs_ref[:, _W:_W + _H * _W] = (
            x_ref[0].reshape(_CIN, _H * _W).astype(jnp.bfloat16))

    x = xs_ref[:, pl.ds(ri * npix, nband)]

    zc = jnp.zeros((_CIN, 1), jnp.bfloat16)
    xl = jnp.concatenate([zc, x[:, :-1]], axis=1) * ml_ref[:]
    xr = jnp.concatenate([x[:, 1:], zc], axis=1) * mr_ref[:]
    stack = jnp.concatenate([xl, x, xr], axis=0)

    rhs = jnp.concatenate(
        [stack[:, dy * _W:dy * _W + npix] for dy in range(3)], axis=0)
    acc = jnp.dot(wk_ref[:], rhs, preferred_element_type=jnp.float32)

    y = jnp.maximum(acc + t_ref[:], 0.0)
    hm = jax.nn.sigmoid(
        jnp.dot(w2_ref[:], y[320:384, :],
                preferred_element_type=jnp.float32) + b2_ref[:])

    dep_ref[0] = y[0:64, :].reshape(64, _R, _W)
    dim_ref[0] = y[64:128, :].reshape(64, _R, _W)
    reg_ref[0] = y[128:192, :].reshape(64, _R, _W)
    rot_ref[0] = y[192:256, :].reshape(64, _R, _W)
    wh_ref[0] = y[256:320, :].reshape(64, _R, _W)
    hm_ref[0] = hm.reshape(3, _R, _W)


def kernel(feat16, feat8, feat4, params):
    del feat16, feat8
    b = feat4.shape[0]

    w_all = jnp.concatenate([params[n]['w1'] for n in _LANE_ORDER], axis=0)
    inv = 1.0 / jnp.sqrt(1.0 + _BN_EPS)
    s = jnp.concatenate([params[n]['gamma'] * inv for n in _LANE_ORDER])
    t = jnp.concatenate(
        [params[n]['b1'] * params[n]['gamma'] * inv + params[n]['beta']
         for n in _LANE_ORDER])
    wk = (jnp.transpose(w_all, (0, 2, 3, 1)).reshape(_COUT, 9 * _CIN)
          * s[:, None]).astype(jnp.bfloat16)
    t = t.reshape(_COUT, 1)
    w2 = params['hm']['w2'][:, :, 0, 0]
    b2 = params['hm']['b2'].reshape(3, 1)

    nband = (_R + 2) * _W
    lane = jnp.arange(nband) % _W
    ml = (lane != 0).astype(jnp.bfloat16).reshape(1, nband)
    mr = (lane != _W - 1).astype(jnp.bfloat16).reshape(1, nband)

    nb = _H // _R
    grid = (b, nb)
    out_shapes = [
        jax.ShapeDtypeStruct((b, 64, _H, _W), jnp.float32),
        jax.ShapeDtypeStruct((b, 64, _H, _W), jnp.float32),
        jax.ShapeDtypeStruct((b, 3, _H, _W), jnp.float32),
        jax.ShapeDtypeStruct((b, 64, _H, _W), jnp.float32),
        jax.ShapeDtypeStruct((b, 64, _H, _W), jnp.float32),
        jax.ShapeDtypeStruct((b, 64, _H, _W), jnp.float32),
    ]
    head_spec = pl.BlockSpec((1, 64, _R, _W), lambda bi, ri: (bi, 0, ri, 0))
    hm_spec = pl.BlockSpec((1, 3, _R, _W), lambda bi, ri: (bi, 0, ri, 0))
    return pl.pallas_call(
        _conv_kernel,
        grid=grid,
        in_specs=[
            pl.BlockSpec((1, _CIN, _H, _W), lambda bi, ri: (bi, 0, 0, 0)),
            pl.BlockSpec((_COUT, 9 * _CIN), lambda bi, ri: (0, 0)),
            pl.BlockSpec((_COUT, 1), lambda bi, ri: (0, 0)),
            pl.BlockSpec((3, _CIN), lambda bi, ri: (0, 0)),
            pl.BlockSpec((3, 1), lambda bi, ri: (0, 0)),
            pl.BlockSpec((1, (_R + 2) * _W), lambda bi, ri: (0, 0)),
            pl.BlockSpec((1, (_R + 2) * _W), lambda bi, ri: (0, 0)),
        ],
        out_specs=[head_spec, head_spec, hm_spec, head_spec, head_spec,
                   head_spec],
        out_shape=out_shapes,
        scratch_shapes=[pltpu.VMEM((_CIN, (_H + 2) * _W), jnp.bfloat16)],
    )(feat4, wk, t, w2, b2, ml, mr)
